# Initial kernel scaffold; baseline (speedup 1.0000x reference)
import jax
import jax.numpy as jnp
from jax.experimental import pallas as pl


def kernel(input, edge_index, W, b):
    raise NotImplementedError("write your pallas kernel here")


# speedup vs baseline: 5.3558x; 5.3558x over previous
import functools

import jax
import jax.numpy as jnp
from jax import lax
from jax.experimental import pallas as pl
from jax.experimental.pallas import tpu as pltpu
from jax.experimental.pallas import tpu_sc as plsc

NC = 2
NS = 16
NW = NC * NS
K = 128


def _fill_zeros_1d(ref):
    for i in range(ref.shape[0] // 16):
        ref[pl.ds(i * 16, 16)] = jnp.zeros((16,), jnp.float32)


def _fill_ones_1d(ref):
    for i in range(ref.shape[0] // 16):
        ref[pl.ds(i * 16, 16)] = jnp.ones((16,), jnp.float32)


def _fill_zeros_2d(ref):
    for r in range(ref.shape[0]):
        for i in range(ref.shape[1] // 16):
            ref[r, pl.ds(i * 16, 16)] = jnp.zeros((16,), jnp.float32)


def _deg_body(col_hbm, out_hbm, idx_v, ones_v, zero_v, deg_sh):
    c = lax.axis_index("c")
    s = lax.axis_index("s")
    w = c * NS + s
    ch = col_hbm.shape[1]
    nacc = deg_sh.shape[0]
    per_tile = nacc // NS
    zr = zero_v.shape[0]

    _fill_ones_1d(ones_v)
    _fill_zeros_1d(zero_v)
    for i in range(per_tile // zr):
        pltpu.sync_copy(zero_v, deg_sh.at[pl.ds(s * per_tile + i * zr, zr)])
    plsc.subcore_barrier()

    pltpu.sync_copy(col_hbm.at[w], idx_v)

    def body(j, _):
        pltpu.sync_copy(ones_v, deg_sh.at[idx_v.at[j]], add=True)
        return 0

    lax.fori_loop(0, ch, body, 0)
    plsc.subcore_barrier()
    pltpu.sync_copy(deg_sh.at[pl.ds(s * per_tile, per_tile)],
                    out_hbm.at[c, pl.ds(s * per_tile, per_tile)])


def _make_deg_kernel(ch, nacc):
    return functools.partial(
        pl.kernel,
        out_type=jax.ShapeDtypeStruct((NC, nacc), jnp.float32),
        mesh=plsc.VectorSubcoreMesh(core_axis_name="c", subcore_axis_name="s"),
        scratch_types=[
            pltpu.VMEM((ch, K), jnp.int32),
            pltpu.VMEM((K,), jnp.float32),
            pltpu.VMEM((64,), jnp.float32),
            pltpu.VMEM_SHARED((nacc,), jnp.float32),
        ],
    )(_deg_body)


def _spmm_body(col_hbm, row_hbm, dinvx_hbm, out_hbm,
               colv, rowv, rows_v, zb, sem, acc_sh):
    c = lax.axis_index("c")
    s = lax.axis_index("s")
    w = c * NS + s
    ch = col_hbm.shape[1]
    nacc = acc_sh.shape[0]
    per_tile = nacc // NS
    zr = zb.shape[0]

    _fill_zeros_2d(zb)
    for i in range(per_tile // zr):
        pltpu.sync_copy(zb, acc_sh.at[pl.ds(s * per_tile + i * zr, zr)])
    plsc.subcore_barrier()

    pltpu.sync_copy(col_hbm.at[w], colv)
    pltpu.sync_copy(row_hbm.at[w], rowv)

    def body(j, _):
        pltpu.async_copy(dinvx_hbm.at[colv.at[j]], rows_v, sem).wait()
        pltpu.sync_copy(rows_v, acc_sh.at[rowv.at[j]], add=True)
        return 0

    lax.fori_loop(0, ch, body, 0)
    plsc.subcore_barrier()
    pltpu.sync_copy(acc_sh.at[pl.ds(s * per_tile, per_tile)],
                    out_hbm.at[c, pl.ds(s * per_tile, per_tile)])


def _make_spmm_kernel(ch, nacc, d):
    return functools.partial(
        pl.kernel,
        out_type=jax.ShapeDtypeStruct((NC, nacc, d), jnp.float32),
        mesh=plsc.VectorSubcoreMesh(core_axis_name="c", subcore_axis_name="s"),
        scratch_types=[
            pltpu.VMEM((ch, K), jnp.int32),
            pltpu.VMEM((ch, K), jnp.int32),
            pltpu.VMEM((K, d), jnp.float32),
            pltpu.VMEM((8, d), jnp.float32),
            pltpu.SemaphoreType.DMA,
            pltpu.VMEM_SHARED((nacc, d), jnp.float32),
        ],
    )(_spmm_body)


def _dense_body(x_ref, w_ref, degp_ref, b_ref, dinvx_ref, base_ref):
    sup = jnp.dot(x_ref[...], w_ref[...], preferred_element_type=jnp.float32)
    d = degp_ref[0, :] + degp_ref[1, :]
    dinvx_ref[...] = sup * (1.0 / d)[:, None]
    base_ref[...] = sup * (5.0 / 6.0) + b_ref[...][None, :]


def _combine_body(base_ref, p_ref, out_ref):
    out_ref[...] = base_ref[...] + (p_ref[0] + p_ref[1]) * (1.0 / 6.0)


def kernel(input, edge_index, W, b):
    n, d_feat = input.shape
    d_out = W.shape[1]
    e = edge_index.shape[1]

    ch = -(-e // (NW * K))
    ep = NW * ch * K
    nacc = -(-(n + 1) // (NS * 64)) * (NS * 64)

    row = edge_index[0]
    col = edge_index[1]
    pad = ep - e
    if pad:
        fill = jnp.full((pad,), n, dtype=jnp.int32)
        row = jnp.concatenate([row, fill])
        col = jnp.concatenate([col, fill])
    row3 = row.reshape(NW, ch, K)
    col3 = col.reshape(NW, ch, K)

    degp = _make_deg_kernel(ch, nacc)(col3)

    bm = 512
    grid = nacc // bm
    dinvx, base = pl.pallas_call(
        _dense_body,
        grid=(grid,),
        in_specs=[
            pl.BlockSpec((bm, d_feat), lambda j: (j, 0)),
            pl.BlockSpec((d_feat, d_out), lambda j: (0, 0)),
            pl.BlockSpec((NC, bm), lambda j: (0, j)),
            pl.BlockSpec((d_out,), lambda j: (0,)),
        ],
        out_specs=[
            pl.BlockSpec((bm, d_out), lambda j: (j, 0)),
            pl.BlockSpec((bm, d_out), lambda j: (j, 0)),
        ],
        out_shape=[
            jax.ShapeDtypeStruct((nacc, d_out), jnp.float32),
            jax.ShapeDtypeStruct((nacc, d_out), jnp.float32),
        ],
    )(input, W, degp, b)

    partials = _make_spmm_kernel(ch, nacc, d_out)(col3, row3, dinvx)

    bm2 = 1000
    out = pl.pallas_call(
        _combine_body,
        grid=(n // bm2,),
        in_specs=[
            pl.BlockSpec((bm2, d_out), lambda j: (j, 0)),
            pl.BlockSpec((NC, bm2, d_out), lambda j: (0, j, 0)),
        ],
        out_specs=pl.BlockSpec((bm2, d_out), lambda j: (j, 0)),
        out_shape=jax.ShapeDtypeStruct((n, d_out), jnp.float32),
    )(base, partials)
    return out
